# baseline (speedup 1.0000x reference)
import jax
import jax.numpy as jnp
from jax.experimental import pallas as pl


def _copy_block(x_ref, o_ref):
    o_ref[...] = x_ref[...]


def kernel(inputs_embeds):
    b, s, h = inputs_embeds.shape
    x = inputs_embeds.reshape(-1, 128)
    rows = x.shape[0]
    block_rows = 1024
    out = pl.pallas_call(
        _copy_block,
        grid=(rows // block_rows,),
        in_specs=[pl.BlockSpec((block_rows, 128), lambda i: (i, 0))],
        out_specs=pl.BlockSpec((block_rows, 128), lambda i: (i, 0)),
        out_shape=jax.ShapeDtypeStruct((rows, 128), x.dtype),
    )(x)
    return out.reshape(b, s, h)
